# Initial kernel scaffold; baseline (speedup 1.0000x reference)
import jax
import jax.numpy as jnp
from jax.experimental import pallas as pl


def kernel(x, edge_index, W1, b1, W2, b2):
    raise NotImplementedError("write your pallas kernel here")


# speedup vs baseline: 37.0021x; 37.0021x over previous
import functools

import jax
import jax.numpy as jnp
from jax import lax
from jax.experimental import pallas as pl
from jax.experimental.pallas import tpu as pltpu
from jax.experimental.pallas import tpu_sc as plsc

NC = 2
NS = 16
NW = NC * NS
G = 128

N_PAD = 10240
ROWS_PER_TILE = N_PAD // NS
ZROWS = 128


def _zero_shared(zer_v, acc_sh, sid, width):
  def zrow(i, _):
    for off in range(0, width, 16):
      zer_v[i, pl.ds(off, 16)] = jnp.zeros((16,), jnp.float32)
    return 0
  lax.fori_loop(0, ZROWS, zrow, 0)
  base = pl.multiple_of(sid * ROWS_PER_TILE, ROWS_PER_TILE)
  for j in range(ROWS_PER_TILE // ZROWS):
    pltpu.sync_copy(zer_v, acc_sh.at[pl.ds(base + j * ZROWS, ZROWS)])


def _sc_degree(dst3):
  groups = dst3.shape[1]
  mesh = plsc.VectorSubcoreMesh(core_axis_name="c", subcore_axis_name="s")

  @functools.partial(
      pl.kernel,
      mesh=mesh,
      out_type=jax.ShapeDtypeStruct((NC, N_PAD), jnp.float32),
      scratch_types=[
          pltpu.VMEM((groups, G), jnp.int32),
          pltpu.VMEM((G,), jnp.float32),
          pltpu.VMEM((ROWS_PER_TILE,), jnp.float32),
          pltpu.VMEM_SHARED((N_PAD,), jnp.float32),
      ],
  )
  def k(dst_hbm, out_hbm, dstv, ones_v, zer_v, acc_sh):
    cid = lax.axis_index("c")
    sid = lax.axis_index("s")
    wid = cid * NS + sid

    for i in range(G // 16):
      ones_v[pl.ds(i * 16, 16)] = jnp.full((16,), 1.0, jnp.float32)
    for i in range(ROWS_PER_TILE // 16):
      zer_v[pl.ds(i * 16, 16)] = jnp.zeros((16,), jnp.float32)
    base = pl.multiple_of(sid * ROWS_PER_TILE, ROWS_PER_TILE)
    pltpu.sync_copy(zer_v, acc_sh.at[pl.ds(base, ROWS_PER_TILE)])
    plsc.subcore_barrier()

    pltpu.sync_copy(dst_hbm.at[wid], dstv)

    def body(g, _):
      pltpu.sync_copy(ones_v, acc_sh.at[dstv.at[g]], add=True)
      return 0
    lax.fori_loop(0, groups, body, 0)

    plsc.subcore_barrier()
    pltpu.sync_copy(acc_sh.at[pl.ds(base, ROWS_PER_TILE)],
                    out_hbm.at[cid, pl.ds(base, ROWS_PER_TILE)])

  return k(dst3)


def _sc_scatter(src3, dst3, table):
  groups = src3.shape[1]
  F = table.shape[1]
  mesh = plsc.VectorSubcoreMesh(core_axis_name="c", subcore_axis_name="s")

  @functools.partial(
      pl.kernel,
      mesh=mesh,
      compiler_params=pltpu.CompilerParams(use_tc_tiling_on_sc=False),
      out_type=jax.ShapeDtypeStruct((NC, N_PAD, F), jnp.float32),
      scratch_types=[
          pltpu.VMEM((groups, G), jnp.int32),
          pltpu.VMEM((groups, G), jnp.int32),
          pltpu.VMEM((G, F), jnp.float32),
          pltpu.VMEM((G, F), jnp.float32),
          pltpu.VMEM((ZROWS, F), jnp.float32),
          pltpu.VMEM_SHARED((N_PAD, F), jnp.float32),
          pltpu.SemaphoreType.DMA,
          pltpu.SemaphoreType.DMA,
      ],
  )
  def k(src_hbm, dst_hbm, tab_hbm, out_hbm,
        srcv, dstv, rows_a, rows_b, zer_v, acc_sh, sem_a, sem_b):
    cid = lax.axis_index("c")
    sid = lax.axis_index("s")
    wid = cid * NS + sid

    _zero_shared(zer_v, acc_sh, sid, F)
    plsc.subcore_barrier()

    pltpu.sync_copy(src_hbm.at[wid], srcv)
    pltpu.sync_copy(dst_hbm.at[wid], dstv)

    pltpu.async_copy(tab_hbm.at[srcv.at[0]], rows_a, sem_a)

    def body(g, _):
      @pl.when(lax.rem(g, 2) == 0)
      def _even():
        @pl.when(g + 1 < groups)
        def _():
          pltpu.async_copy(tab_hbm.at[srcv.at[g + 1]], rows_b, sem_b)
        pltpu.make_async_copy(tab_hbm.at[srcv.at[g]], rows_a, sem_a).wait()
        pltpu.sync_copy(rows_a, acc_sh.at[dstv.at[g]], add=True)

      @pl.when(lax.rem(g, 2) == 1)
      def _odd():
        @pl.when(g + 1 < groups)
        def _():
          pltpu.async_copy(tab_hbm.at[srcv.at[g + 1]], rows_a, sem_a)
        pltpu.make_async_copy(tab_hbm.at[srcv.at[g]], rows_b, sem_b).wait()
        pltpu.sync_copy(rows_b, acc_sh.at[dstv.at[g]], add=True)
      return 0

    lax.fori_loop(0, groups, body, 0)

    plsc.subcore_barrier()
    base = pl.multiple_of(sid * ROWS_PER_TILE, ROWS_PER_TILE)
    pltpu.sync_copy(acc_sh.at[pl.ds(base, ROWS_PER_TILE)],
                    out_hbm.at[cid, pl.ds(base, ROWS_PER_TILE)])

  return k(src3, dst3, table)


def _tc_lin1(x, W1, degp):
  n = x.shape[0]
  h = W1.shape[1]

  def body(x_ref, w_ref, deg_ref, y_ref, dinv_ref):
    deg = deg_ref[0] + deg_ref[1]
    dinv = jnp.where(deg > 0, lax.rsqrt(deg), 0.0)
    lin = jnp.dot(x_ref[...], w_ref[...], preferred_element_type=jnp.float32)
    y_ref[...] = lin * dinv
    dinv_ref[...] = dinv

  return pl.pallas_call(
      body,
      out_shape=[jax.ShapeDtypeStruct((n, h), jnp.float32),
                 jax.ShapeDtypeStruct((n, 1), jnp.float32)],
  )(x, W1, degp)


def _tc_lin2(accp, dinv, b1, W2p):
  n = accp.shape[1]
  cp = W2p.shape[1]

  def body(a_ref, dinv_ref, b_ref, w_ref, y_ref):
    a = a_ref[0] + a_ref[1]
    hid = jnp.maximum(a * dinv_ref[...] + b_ref[...], 0.0)
    lin = jnp.dot(hid, w_ref[...], preferred_element_type=jnp.float32)
    y_ref[...] = lin * dinv_ref[...]

  return pl.pallas_call(
      body,
      out_shape=jax.ShapeDtypeStruct((n, cp), jnp.float32),
  )(accp, dinv, b1, W2p)


def _tc_out(accp, dinv, b2):
  n, c = accp.shape[1], accp.shape[2]

  def body(a_ref, dinv_ref, b_ref, o_ref):
    o = (a_ref[0] + a_ref[1]) * dinv_ref[...] + b_ref[...]
    m = jnp.max(o, axis=1, keepdims=True)
    s = o - m
    lse = jnp.log(jnp.sum(jnp.exp(s), axis=1, keepdims=True))
    o_ref[...] = s - lse

  return pl.pallas_call(
      body,
      out_shape=jax.ShapeDtypeStruct((n, c), jnp.float32),
  )(accp, dinv, b2)


def kernel(x, edge_index, W1, b1, W2, b2):
  n, d = x.shape
  h = W1.shape[1]
  c = W2.shape[1]
  cp = 48

  loop = jnp.arange(n, dtype=edge_index.dtype)
  src = jnp.concatenate([edge_index[0], loop])
  dst = jnp.concatenate([edge_index[1], loop])
  e = src.shape[0]
  chunk = NW * G
  ep = chunk * ((e + chunk - 1) // chunk)
  src = jnp.concatenate([src, jnp.zeros((ep - e,), edge_index.dtype)])
  dst = jnp.concatenate([dst, jnp.full((ep - e,), n, edge_index.dtype)])
  groups = ep // chunk
  src3 = src.reshape(NW, groups, G)
  dst3 = dst.reshape(NW, groups, G)

  W2p = jnp.zeros((h, cp), jnp.float32).at[:, :c].set(W2)

  degp = _sc_degree(dst3)
  y1, dinv = _tc_lin1(x, W1, degp[:, :n].reshape(NC, n, 1))
  acc1 = _sc_scatter(src3, dst3, y1)
  y2 = _tc_lin2(acc1[:, :n], dinv, b1.reshape(1, h), W2p)
  acc2 = _sc_scatter(src3, dst3, y2)
  return _tc_out(acc2[:, :n, :c], dinv, b2.reshape(1, c))


# speedup vs baseline: 42.3907x; 1.1456x over previous
import functools

import jax
import jax.numpy as jnp
from jax import lax
from jax.experimental import pallas as pl
from jax.experimental.pallas import tpu as pltpu
from jax.experimental.pallas import tpu_sc as plsc

NC = 2
NS = 16
NW = NC * NS
G = 128

N_PAD = 10240
ROWS_PER_TILE = N_PAD // NS
ZROWS = 128


def _zero_shared(zer_v, acc_sh, sid, width):
  def zrow(i, _):
    for off in range(0, width, 16):
      zer_v[i, pl.ds(off, 16)] = jnp.zeros((16,), jnp.float32)
    return 0
  lax.fori_loop(0, ZROWS, zrow, 0)
  base = pl.multiple_of(sid * ROWS_PER_TILE, ROWS_PER_TILE)
  for j in range(ROWS_PER_TILE // ZROWS):
    pltpu.sync_copy(zer_v, acc_sh.at[pl.ds(base + j * ZROWS, ZROWS)])


def _sc_degree(dst3):
  groups = dst3.shape[1]
  mesh = plsc.VectorSubcoreMesh(core_axis_name="c", subcore_axis_name="s")

  @functools.partial(
      pl.kernel,
      mesh=mesh,
      out_type=jax.ShapeDtypeStruct((NC, N_PAD), jnp.float32),
      scratch_types=[
          pltpu.VMEM((groups, G), jnp.int32),
          pltpu.VMEM((G,), jnp.float32),
          pltpu.VMEM((ROWS_PER_TILE,), jnp.float32),
          pltpu.VMEM_SHARED((N_PAD,), jnp.float32),
          pltpu.SemaphoreType.DMA,
      ],
  )
  def k(dst_hbm, out_hbm, dstv, ones_v, zer_v, acc_sh, sem):
    cid = lax.axis_index("c")
    sid = lax.axis_index("s")
    wid = cid * NS + sid

    for i in range(G // 16):
      ones_v[pl.ds(i * 16, 16)] = jnp.full((16,), 1.0, jnp.float32)
    for i in range(ROWS_PER_TILE // 16):
      zer_v[pl.ds(i * 16, 16)] = jnp.zeros((16,), jnp.float32)
    base = pl.multiple_of(sid * ROWS_PER_TILE, ROWS_PER_TILE)
    pltpu.sync_copy(zer_v, acc_sh.at[pl.ds(base, ROWS_PER_TILE)])
    plsc.subcore_barrier()

    pltpu.sync_copy(dst_hbm.at[wid], dstv)

    def body(g, _):
      pltpu.async_copy(ones_v, acc_sh.at[dstv.at[g]], sem, add=True)
      return 0
    lax.fori_loop(0, groups, body, 0)

    def drain(g, _):
      pltpu.make_async_copy(ones_v, acc_sh.at[dstv.at[0]], sem).wait()
      return 0
    lax.fori_loop(0, groups, drain, 0)

    plsc.subcore_barrier()
    pltpu.sync_copy(acc_sh.at[pl.ds(base, ROWS_PER_TILE)],
                    out_hbm.at[cid, pl.ds(base, ROWS_PER_TILE)])

  return k(dst3)


def _sc_scatter(src3, dst3, table):
  groups = src3.shape[1]
  F = table.shape[1]
  mesh = plsc.VectorSubcoreMesh(core_axis_name="c", subcore_axis_name="s")

  R = 8
  A = 4
  assert groups >= R

  @functools.partial(
      pl.kernel,
      mesh=mesh,
      compiler_params=pltpu.CompilerParams(use_tc_tiling_on_sc=False),
      out_type=jax.ShapeDtypeStruct((NC, N_PAD, F), jnp.float32),
      scratch_types=[
          pltpu.VMEM((groups, G), jnp.int32),
          pltpu.VMEM((groups, G), jnp.int32),
          pltpu.VMEM((R, G, F), jnp.float32),
          pltpu.VMEM((ZROWS, F), jnp.float32),
          pltpu.VMEM_SHARED((N_PAD, F), jnp.float32),
          pltpu.SemaphoreType.DMA((R,)),
          pltpu.SemaphoreType.DMA((R,)),
      ],
  )
  def k(src_hbm, dst_hbm, tab_hbm, out_hbm,
        srcv, dstv, rows, zer_v, acc_sh, sem_g, sem_s):
    cid = lax.axis_index("c")
    sid = lax.axis_index("s")
    wid = cid * NS + sid

    _zero_shared(zer_v, acc_sh, sid, F)
    plsc.subcore_barrier()

    pltpu.sync_copy(src_hbm.at[wid], srcv)
    pltpu.sync_copy(dst_hbm.at[wid], dstv)

    for a in range(A):
      pltpu.async_copy(tab_hbm.at[srcv.at[a]], rows.at[a], sem_g.at[a])

    def body(g, _):
      @pl.when(g + A < groups)
      def _pref():
        bp = lax.rem(g + A, R)
        @pl.when(g + A >= R)
        def _wait_s():
          pltpu.make_async_copy(
              rows.at[bp], acc_sh.at[dstv.at[0]], sem_s.at[bp]).wait()
        pltpu.async_copy(tab_hbm.at[srcv.at[g + A]], rows.at[bp],
                         sem_g.at[bp])

      b = lax.rem(g, R)
      pltpu.make_async_copy(tab_hbm.at[srcv.at[g]], rows.at[b],
                            sem_g.at[b]).wait()
      pltpu.async_copy(rows.at[b], acc_sh.at[dstv.at[g]], sem_s.at[b],
                       add=True)
      return 0

    lax.fori_loop(0, groups, body, 0)

    for i in range(R):
      b = (groups - R + i) % R
      pltpu.make_async_copy(rows.at[b], acc_sh.at[dstv.at[0]],
                            sem_s.at[b]).wait()

    plsc.subcore_barrier()
    base = pl.multiple_of(sid * ROWS_PER_TILE, ROWS_PER_TILE)
    pltpu.sync_copy(acc_sh.at[pl.ds(base, ROWS_PER_TILE)],
                    out_hbm.at[cid, pl.ds(base, ROWS_PER_TILE)])

  return k(src3, dst3, table)


def _tc_lin1(x, W1, degp):
  n = x.shape[0]
  h = W1.shape[1]

  def body(x_ref, w_ref, deg_ref, y_ref, dinv_ref):
    deg = deg_ref[0] + deg_ref[1]
    dinv = jnp.where(deg > 0, lax.rsqrt(deg), 0.0)
    lin = jnp.dot(x_ref[...], w_ref[...], preferred_element_type=jnp.float32)
    y_ref[...] = lin * dinv
    dinv_ref[...] = dinv

  return pl.pallas_call(
      body,
      out_shape=[jax.ShapeDtypeStruct((n, h), jnp.float32),
                 jax.ShapeDtypeStruct((n, 1), jnp.float32)],
  )(x, W1, degp)


def _tc_lin2(accp, dinv, b1, W2p):
  n = accp.shape[1]
  cp = W2p.shape[1]

  def body(a_ref, dinv_ref, b_ref, w_ref, y_ref):
    a = a_ref[0] + a_ref[1]
    hid = jnp.maximum(a * dinv_ref[...] + b_ref[...], 0.0)
    lin = jnp.dot(hid, w_ref[...], preferred_element_type=jnp.float32)
    y_ref[...] = lin * dinv_ref[...]

  return pl.pallas_call(
      body,
      out_shape=jax.ShapeDtypeStruct((n, cp), jnp.float32),
  )(accp, dinv, b1, W2p)


def _tc_out(accp, dinv, b2):
  n, c = accp.shape[1], accp.shape[2]

  def body(a_ref, dinv_ref, b_ref, o_ref):
    o = (a_ref[0] + a_ref[1]) * dinv_ref[...] + b_ref[...]
    m = jnp.max(o, axis=1, keepdims=True)
    s = o - m
    lse = jnp.log(jnp.sum(jnp.exp(s), axis=1, keepdims=True))
    o_ref[...] = s - lse

  return pl.pallas_call(
      body,
      out_shape=jax.ShapeDtypeStruct((n, c), jnp.float32),
  )(accp, dinv, b2)


def kernel(x, edge_index, W1, b1, W2, b2):
  n, d = x.shape
  h = W1.shape[1]
  c = W2.shape[1]
  cp = 48

  loop = jnp.arange(n, dtype=edge_index.dtype)
  src = jnp.concatenate([edge_index[0], loop])
  dst = jnp.concatenate([edge_index[1], loop])
  e = src.shape[0]
  chunk = NW * G
  ep = chunk * ((e + chunk - 1) // chunk)
  src = jnp.concatenate([src, jnp.zeros((ep - e,), edge_index.dtype)])
  dst = jnp.concatenate([dst, jnp.full((ep - e,), n, edge_index.dtype)])
  groups = ep // chunk
  src3 = src.reshape(NW, groups, G)
  dst3 = dst.reshape(NW, groups, G)

  W2p = jnp.zeros((h, cp), jnp.float32).at[:, :c].set(W2)

  degp = _sc_degree(dst3)
  y1, dinv = _tc_lin1(x, W1, degp[:, :n].reshape(NC, n, 1))
  acc1 = _sc_scatter(src3, dst3, y1)
  y2 = _tc_lin2(acc1[:, :n], dinv, b1.reshape(1, h), W2p)
  acc2 = _sc_scatter(src3, dst3, y2)
  return _tc_out(acc2[:, :n, :c], dinv, b2.reshape(1, c))
